# Initial kernel scaffold; baseline (speedup 1.0000x reference)
import jax
import jax.numpy as jnp
from jax.experimental import pallas as pl


def kernel(x, edge_index, edge_weight, W1, b1, W2, b2, W3, b3):
    raise NotImplementedError("write your pallas kernel here")


# speedup vs baseline: 15.5292x; 15.5292x over previous
import functools

import jax
import jax.numpy as jnp
from jax import lax
from jax.experimental import pallas as pl
from jax.experimental.pallas import tpu as pltpu
from jax.experimental.pallas import tpu_sc as plsc

_NC = 2
_NS = 16
_L = 16


def _sc_edge_aggregate(h, src_r, dst_r, w_r, zeros_nd):
    n_nodes, d = h.shape
    n_rows_total, c_chunk = src_r.shape
    nw = _NC * _NS
    stage = 8
    n_outer = n_rows_total // (nw * stage)
    rpt_a = ((n_nodes // _NS) + 7) // 8 * 8
    rpt_b = n_nodes - (_NS - 1) * rpt_a

    mesh = plsc.VectorSubcoreMesh(core_axis_name="c", subcore_axis_name="s")

    @functools.partial(
        pl.kernel,
        out_type=jax.ShapeDtypeStruct((_NC, n_nodes, d), jnp.float32),
        mesh=mesh,
        scratch_types=[
            pltpu.VMEM((stage, c_chunk), jnp.int32),
            pltpu.VMEM((stage, c_chunk), jnp.int32),
            pltpu.VMEM((stage, c_chunk), jnp.float32),
            pltpu.VMEM((2, c_chunk, d), jnp.float32),
            pltpu.VMEM_SHARED((n_nodes, d), jnp.float32),
            pltpu.SemaphoreType.DMA,
            pltpu.SemaphoreType.DMA,
        ],
        compiler_params=pltpu.CompilerParams(use_tc_tiling_on_sc=False),
    )
    def k(h_hbm, src_hbm, dst_hbm, w_hbm, z_hbm, out_hbm,
          src_v, dst_v, w_v, rows_v, acc, sem0, sem1):
        cid = lax.axis_index("c")
        sid = lax.axis_index("s")

        @pl.when(sid < _NS - 1)
        def _():
            pltpu.sync_copy(z_hbm.at[pl.ds(sid * rpt_a, rpt_a)],
                            acc.at[pl.ds(sid * rpt_a, rpt_a)])

        @pl.when(sid == _NS - 1)
        def _():
            pltpu.sync_copy(z_hbm.at[pl.ds((_NS - 1) * rpt_a, rpt_b)],
                            acc.at[pl.ds((_NS - 1) * rpt_a, rpt_b)])

        plsc.subcore_barrier()

        wid = cid * _NS + sid
        sems = [sem0, sem1]

        splat_dnums = lax.GatherDimensionNumbers(
            offset_dims=(), collapsed_slice_dims=(0,), start_index_map=(0,))

        def scale_rows(buf, j):
            w16 = None
            cur_w0 = -1
            for r in range(c_chunk):
                w0 = min((r // _L) * _L, c_chunk - _L)
                if w0 != cur_w0:
                    w16 = w_v[j, pl.ds(w0, _L)]
                    cur_w0 = w0
                wsplat = lax.gather(
                    w16, jnp.full((_L, 1), r - w0, jnp.int32),
                    dimension_numbers=splat_dnums, slice_sizes=(1,),
                    mode=lax.GatherScatterMode.PROMISE_IN_BOUNDS)
                rows_v[buf, r, :] = rows_v[buf, r, :] * wsplat

        def outer(o, carry):
            row0 = (o * nw + wid) * stage
            pltpu.sync_copy(src_hbm.at[pl.ds(row0, stage)], src_v)
            pltpu.sync_copy(dst_hbm.at[pl.ds(row0, stage)], dst_v)
            pltpu.sync_copy(w_hbm.at[pl.ds(row0, stage)], w_v)

            cp0 = pltpu.async_copy(h_hbm.at[src_v.at[0]], rows_v.at[0], sem0)
            cp0.wait()
            for jj in range(stage):
                buf = jj % 2
                if jj + 1 < stage:
                    nxt = pltpu.async_copy(
                        h_hbm.at[src_v.at[jj + 1]],
                        rows_v.at[(jj + 1) % 2],
                        sems[(jj + 1) % 2])
                scale_rows(buf, jj)
                pltpu.sync_copy(rows_v.at[buf], acc.at[dst_v.at[jj]], add=True)
                if jj + 1 < stage:
                    nxt.wait()
            return carry

        lax.fori_loop(0, n_outer, outer, 0)

        plsc.subcore_barrier()

        @pl.when(sid < _NS - 1)
        def _():
            pltpu.sync_copy(acc.at[pl.ds(sid * rpt_a, rpt_a)],
                            out_hbm.at[cid, pl.ds(sid * rpt_a, rpt_a)])

        @pl.when(sid == _NS - 1)
        def _():
            pltpu.sync_copy(acc.at[pl.ds((_NS - 1) * rpt_a, rpt_b)],
                            out_hbm.at[cid, pl.ds((_NS - 1) * rpt_a, rpt_b)])

    return k(h, src_r, dst_r, w_r, zeros_nd)


_BLK = 2000


def _tc_in_proj(x, w1):
    n, kdim = x.shape
    d = w1.shape[0]

    def body(x_ref, w_ref, o_ref):
        o_ref[...] = lax.dot_general(
            x_ref[...], w_ref[...], (((1,), (1,)), ((), ())),
            preferred_element_type=jnp.float32)

    return pl.pallas_call(
        body,
        grid=(n // _BLK,),
        in_specs=[
            pl.BlockSpec((_BLK, kdim), lambda i: (i, 0)),
            pl.BlockSpec((d, kdim), lambda i: (0, 0)),
        ],
        out_specs=pl.BlockSpec((_BLK, d), lambda i: (i, 0)),
        out_shape=jax.ShapeDtypeStruct((n, d), jnp.float32),
    )(x, w1)


def _tc_combine_linear(parts, b, w):
    _, n, d = parts.shape
    d_out = w.shape[0]

    def body(p_ref, b_ref, w_ref, o_ref):
        t = jax.nn.relu(p_ref[0] + p_ref[1] + b_ref[...])
        o_ref[...] = lax.dot_general(
            t, w_ref[...], (((1,), (1,)), ((), ())),
            preferred_element_type=jnp.float32)

    return pl.pallas_call(
        body,
        grid=(n // _BLK,),
        in_specs=[
            pl.BlockSpec((2, _BLK, d), lambda i: (0, i, 0)),
            pl.BlockSpec((1, d), lambda i: (0, 0)),
            pl.BlockSpec((d_out, d), lambda i: (0, 0)),
        ],
        out_specs=pl.BlockSpec((_BLK, d_out), lambda i: (i, 0)),
        out_shape=jax.ShapeDtypeStruct((n, d_out), jnp.float32),
    )(parts, b.reshape(1, d), w)


def _tc_final(parts, b2, w3, b3):
    _, n, d = parts.shape
    d_out = w3.shape[0]

    def body(p_ref, b2_ref, w_ref, b3_ref, o_ref):
        t = jax.nn.relu(p_ref[0] + p_ref[1] + b2_ref[...])
        logits = lax.dot_general(
            t, w_ref[...], (((1,), (1,)), ((), ())),
            preferred_element_type=jnp.float32) + b3_ref[...]
        m = jnp.max(logits, axis=1, keepdims=True)
        e = jnp.exp(logits - m)
        o_ref[...] = e / jnp.sum(e, axis=1, keepdims=True)

    return pl.pallas_call(
        body,
        grid=(n // _BLK,),
        in_specs=[
            pl.BlockSpec((2, _BLK, d), lambda i: (0, i, 0)),
            pl.BlockSpec((1, d), lambda i: (0, 0)),
            pl.BlockSpec((d_out, d), lambda i: (0, 0)),
            pl.BlockSpec((1, d_out), lambda i: (0, 0)),
        ],
        out_specs=pl.BlockSpec((_BLK, d_out), lambda i: (i, 0)),
        out_shape=jax.ShapeDtypeStruct((n, d_out), jnp.float32),
    )(parts, b2.reshape(1, d), w3, b3.reshape(1, d_out))


def kernel(x, edge_index, edge_weight, W1, b1, W2, b2, W3, b3):
    n = x.shape[0]
    e = edge_weight.shape[0]
    c_chunk = 100
    src_r = edge_index[0].reshape(e // c_chunk, c_chunk)
    dst_r = edge_index[1].reshape(e // c_chunk, c_chunk)
    w_r = edge_weight.reshape(e // c_chunk, c_chunk)
    zeros_nd = jnp.zeros((n, W1.shape[0]), jnp.float32)

    h1 = _tc_in_proj(x, W1)
    p1 = _sc_edge_aggregate(h1, src_r, dst_r, w_r, zeros_nd)
    h2 = _tc_combine_linear(p1, b1, W2)
    p2 = _sc_edge_aggregate(h2, src_r, dst_r, w_r, zeros_nd)
    return _tc_final(p2, b2, W3, b3)


# speedup vs baseline: 30.3590x; 1.9550x over previous
import functools

import jax
import jax.numpy as jnp
from jax import lax
from jax.experimental import pallas as pl
from jax.experimental.pallas import tpu as pltpu
from jax.experimental.pallas import tpu_sc as plsc

_NC = 2
_NS = 16
_L = 16


def _sc_edge_aggregate(h, src_r, dst_r, w_r, zeros_nd):
    n_nodes, d = h.shape
    n_rows_total, c_chunk = src_r.shape
    nw = _NC * _NS
    stage = 40
    nbuf = 4
    group = stage // nbuf
    n_outer = n_rows_total // (nw * stage)
    rpt_a = ((n_nodes // _NS) + 7) // 8 * 8
    rpt_b = n_nodes - (_NS - 1) * rpt_a

    mesh = plsc.VectorSubcoreMesh(core_axis_name="c", subcore_axis_name="s")

    @functools.partial(
        pl.kernel,
        out_type=jax.ShapeDtypeStruct((_NC, n_nodes, d), jnp.float32),
        mesh=mesh,
        scratch_types=[
            pltpu.VMEM((stage, c_chunk), jnp.int32),
            pltpu.VMEM((stage, c_chunk), jnp.int32),
            pltpu.VMEM((stage, c_chunk), jnp.float32),
            pltpu.VMEM((nbuf, c_chunk, d), jnp.float32),
            pltpu.VMEM_SHARED((n_nodes, d), jnp.float32),
            [pltpu.SemaphoreType.DMA] * nbuf,
        ],
        compiler_params=pltpu.CompilerParams(use_tc_tiling_on_sc=False),
    )
    def k(h_hbm, src_hbm, dst_hbm, w_hbm, z_hbm, out_hbm,
          src_v, dst_v, w_v, rows_v, acc, sems):
        cid = lax.axis_index("c")
        sid = lax.axis_index("s")

        @pl.when(sid < _NS - 1)
        def _():
            pltpu.sync_copy(z_hbm.at[pl.ds(sid * rpt_a, rpt_a)],
                            acc.at[pl.ds(sid * rpt_a, rpt_a)])

        @pl.when(sid == _NS - 1)
        def _():
            pltpu.sync_copy(z_hbm.at[pl.ds((_NS - 1) * rpt_a, rpt_b)],
                            acc.at[pl.ds((_NS - 1) * rpt_a, rpt_b)])

        plsc.subcore_barrier()

        wid = cid * _NS + sid

        splat_dnums = lax.GatherDimensionNumbers(
            offset_dims=(), collapsed_slice_dims=(0,), start_index_map=(0,))

        def scale_rows(buf, j):
            w16 = None
            cur_w0 = -1
            for r in range(c_chunk):
                w0 = min((r // _L) * _L, c_chunk - _L)
                if w0 != cur_w0:
                    w16 = w_v[j, pl.ds(w0, _L)]
                    cur_w0 = w0
                wsplat = lax.gather(
                    w16, jnp.full((_L, 1), r - w0, jnp.int32),
                    dimension_numbers=splat_dnums, slice_sizes=(1,),
                    mode=lax.GatherScatterMode.PROMISE_IN_BOUNDS)
                rows_v[buf, r, :] = rows_v[buf, r, :] * wsplat

        def issue_gather(j, buf):
            pltpu.async_copy(h_hbm.at[src_v.at[j]], rows_v.at[buf], sems[buf])

        def wait_gather(j, buf):
            pltpu.make_async_copy(h_hbm.at[src_v.at[j]], rows_v.at[buf],
                                  sems[buf]).wait()

        def outer(o, carry):
            row0 = (o * nw + wid) * stage
            pltpu.sync_copy(src_hbm.at[pl.ds(row0, stage)], src_v)
            pltpu.sync_copy(dst_hbm.at[pl.ds(row0, stage)], dst_v)
            pltpu.sync_copy(w_hbm.at[pl.ds(row0, stage)], w_v)

            for j0 in range(nbuf - 1):
                issue_gather(j0, j0)

            def inner(g, carry2):
                for k_slot in range(nbuf):
                    j = g * nbuf + k_slot
                    wait_gather(j, k_slot)
                    scale_rows(k_slot, j)
                    pltpu.sync_copy(rows_v.at[k_slot],
                                    acc.at[dst_v.at[j]], add=True)

                    @pl.when(j + nbuf - 1 < stage)
                    def _():
                        issue_gather(j + nbuf - 1, (k_slot + nbuf - 1) % nbuf)
                return carry2

            lax.fori_loop(0, group, inner, 0)
            return carry

        lax.fori_loop(0, n_outer, outer, 0)

        plsc.subcore_barrier()

        @pl.when(sid < _NS - 1)
        def _():
            pltpu.sync_copy(acc.at[pl.ds(sid * rpt_a, rpt_a)],
                            out_hbm.at[cid, pl.ds(sid * rpt_a, rpt_a)])

        @pl.when(sid == _NS - 1)
        def _():
            pltpu.sync_copy(acc.at[pl.ds((_NS - 1) * rpt_a, rpt_b)],
                            out_hbm.at[cid, pl.ds((_NS - 1) * rpt_a, rpt_b)])

    return k(h, src_r, dst_r, w_r, zeros_nd)


_BLK = 2000


def _tc_in_proj(x, w1):
    n, kdim = x.shape
    d = w1.shape[0]

    def body(x_ref, w_ref, o_ref):
        o_ref[...] = lax.dot_general(
            x_ref[...], w_ref[...], (((1,), (1,)), ((), ())),
            preferred_element_type=jnp.float32)

    return pl.pallas_call(
        body,
        grid=(n // _BLK,),
        in_specs=[
            pl.BlockSpec((_BLK, kdim), lambda i: (i, 0)),
            pl.BlockSpec((d, kdim), lambda i: (0, 0)),
        ],
        out_specs=pl.BlockSpec((_BLK, d), lambda i: (i, 0)),
        out_shape=jax.ShapeDtypeStruct((n, d), jnp.float32),
    )(x, w1)


def _tc_combine_linear(parts, b, w):
    _, n, d = parts.shape
    d_out = w.shape[0]

    def body(p_ref, b_ref, w_ref, o_ref):
        t = jax.nn.relu(p_ref[0] + p_ref[1] + b_ref[...])
        o_ref[...] = lax.dot_general(
            t, w_ref[...], (((1,), (1,)), ((), ())),
            preferred_element_type=jnp.float32)

    return pl.pallas_call(
        body,
        grid=(n // _BLK,),
        in_specs=[
            pl.BlockSpec((2, _BLK, d), lambda i: (0, i, 0)),
            pl.BlockSpec((1, d), lambda i: (0, 0)),
            pl.BlockSpec((d_out, d), lambda i: (0, 0)),
        ],
        out_specs=pl.BlockSpec((_BLK, d_out), lambda i: (i, 0)),
        out_shape=jax.ShapeDtypeStruct((n, d_out), jnp.float32),
    )(parts, b.reshape(1, d), w)


def _tc_final(parts, b2, w3, b3):
    _, n, d = parts.shape
    d_out = w3.shape[0]

    def body(p_ref, b2_ref, w_ref, b3_ref, o_ref):
        t = jax.nn.relu(p_ref[0] + p_ref[1] + b2_ref[...])
        logits = lax.dot_general(
            t, w_ref[...], (((1,), (1,)), ((), ())),
            preferred_element_type=jnp.float32) + b3_ref[...]
        m = jnp.max(logits, axis=1, keepdims=True)
        e = jnp.exp(logits - m)
        o_ref[...] = e / jnp.sum(e, axis=1, keepdims=True)

    return pl.pallas_call(
        body,
        grid=(n // _BLK,),
        in_specs=[
            pl.BlockSpec((2, _BLK, d), lambda i: (0, i, 0)),
            pl.BlockSpec((1, d), lambda i: (0, 0)),
            pl.BlockSpec((d_out, d), lambda i: (0, 0)),
            pl.BlockSpec((1, d_out), lambda i: (0, 0)),
        ],
        out_specs=pl.BlockSpec((_BLK, d_out), lambda i: (i, 0)),
        out_shape=jax.ShapeDtypeStruct((n, d_out), jnp.float32),
    )(parts, b2.reshape(1, d), w3, b3.reshape(1, d_out))


def kernel(x, edge_index, edge_weight, W1, b1, W2, b2, W3, b3):
    n = x.shape[0]
    e = edge_weight.shape[0]
    c_chunk = 125
    src_r = edge_index[0].reshape(e // c_chunk, c_chunk)
    dst_r = edge_index[1].reshape(e // c_chunk, c_chunk)
    w_r = edge_weight.reshape(e // c_chunk, c_chunk)
    zeros_nd = jnp.zeros((n, W1.shape[0]), jnp.float32)

    h1 = _tc_in_proj(x, W1)
    p1 = _sc_edge_aggregate(h1, src_r, dst_r, w_r, zeros_nd)
    h2 = _tc_combine_linear(p1, b1, W2)
    p2 = _sc_edge_aggregate(h2, src_r, dst_r, w_r, zeros_nd)
    return _tc_final(p2, b2, W3, b3)


# speedup vs baseline: 42.6470x; 1.4048x over previous
import functools

import jax
import jax.numpy as jnp
from jax import lax
from jax.experimental import pallas as pl
from jax.experimental.pallas import tpu as pltpu
from jax.experimental.pallas import tpu_sc as plsc

_NC = 2
_NS = 16
_L = 16


def _sc_edge_aggregate(h, edge_r, w_r, zeros_nd):
    n_nodes, d = h.shape
    n_rows_total, c_chunk = w_r.shape
    nw = _NC * _NS
    stage = 40
    nbuf = 5
    group = stage // nbuf
    total_blocks = n_rows_total // stage
    n_base = total_blocks // nw
    extra = total_blocks % nw
    rpt_a = ((n_nodes // _NS) + 7) // 8 * 8
    rpt_b = n_nodes - (_NS - 1) * rpt_a

    mesh = plsc.VectorSubcoreMesh(core_axis_name="c", subcore_axis_name="s")

    @functools.partial(
        pl.kernel,
        out_type=jax.ShapeDtypeStruct((_NC, n_nodes, d), jnp.float32),
        mesh=mesh,
        scratch_types=[
            pltpu.VMEM((stage, c_chunk), jnp.int32),
            pltpu.VMEM((stage, c_chunk), jnp.int32),
            pltpu.VMEM((stage, c_chunk), jnp.float32),
            pltpu.VMEM((nbuf, c_chunk, d), jnp.float32),
            pltpu.VMEM_SHARED((n_nodes, d), jnp.float32),
            [pltpu.SemaphoreType.DMA] * nbuf,
            [pltpu.SemaphoreType.DMA] * nbuf,
        ],
        compiler_params=pltpu.CompilerParams(use_tc_tiling_on_sc=False),
    )
    def k(h_hbm, edge_hbm, w_hbm, z_hbm, out_hbm,
          src_v, dst_v, w_v, rows_v, acc, sems, ssems):
        cid = lax.axis_index("c")
        sid = lax.axis_index("s")

        @pl.when(sid < _NS - 1)
        def _():
            pltpu.sync_copy(z_hbm.at[pl.ds(sid * rpt_a, rpt_a)],
                            acc.at[pl.ds(sid * rpt_a, rpt_a)])

        @pl.when(sid == _NS - 1)
        def _():
            pltpu.sync_copy(z_hbm.at[pl.ds((_NS - 1) * rpt_a, rpt_b)],
                            acc.at[pl.ds((_NS - 1) * rpt_a, rpt_b)])

        plsc.subcore_barrier()

        wid = cid * _NS + sid
        n_outer_w = n_base + jnp.where(wid < extra, 1, 0)

        splat_dnums = lax.GatherDimensionNumbers(
            offset_dims=(), collapsed_slice_dims=(0,), start_index_map=(0,))

        def scale_rows(buf, j):
            w16 = None
            cur_w0 = -1
            for r in range(c_chunk):
                w0 = min((r // _L) * _L, c_chunk - _L)
                if w0 != cur_w0:
                    w16 = w_v[j, pl.ds(w0, _L)]
                    cur_w0 = w0
                wsplat = lax.gather(
                    w16, jnp.full((_L, 1), r - w0, jnp.int32),
                    dimension_numbers=splat_dnums, slice_sizes=(1,),
                    mode=lax.GatherScatterMode.PROMISE_IN_BOUNDS)
                rows_v[buf, r, :] = rows_v[buf, r, :] * wsplat

        def issue_gather(j, buf):
            pltpu.async_copy(h_hbm.at[src_v.at[j]], rows_v.at[buf], sems[buf])

        def wait_gather(j, buf):
            pltpu.make_async_copy(h_hbm.at[src_v.at[j]], rows_v.at[buf],
                                  sems[buf]).wait()

        def outer(o, carry):
            row0 = (o * nw + wid) * stage
            pltpu.sync_copy(edge_hbm.at[0, pl.ds(row0, stage)], src_v)
            pltpu.sync_copy(edge_hbm.at[1, pl.ds(row0, stage)], dst_v)
            pltpu.sync_copy(w_hbm.at[pl.ds(row0, stage)], w_v)

            for j0 in range(nbuf - 1):
                issue_gather(j0, j0)

            def inner(g, carry2):
                for k_slot in range(nbuf):
                    j = g * nbuf + k_slot
                    wait_gather(j, k_slot)
                    scale_rows(k_slot, j)
                    pltpu.async_copy(rows_v.at[k_slot],
                                     acc.at[dst_v.at[j]], ssems[k_slot],
                                     add=True)

                    @pl.when(j > 0)
                    def _():
                        pltpu.make_async_copy(
                            rows_v.at[(k_slot + nbuf - 1) % nbuf],
                            acc.at[dst_v.at[j - 1]],
                            ssems[(k_slot + nbuf - 1) % nbuf]).wait()

                    @pl.when(j + nbuf - 1 < stage)
                    def _():
                        issue_gather(j + nbuf - 1, (k_slot + nbuf - 1) % nbuf)
                return carry2

            lax.fori_loop(0, group, inner, 0)
            pltpu.make_async_copy(rows_v.at[(stage - 1) % nbuf],
                                  acc.at[dst_v.at[stage - 1]],
                                  ssems[(stage - 1) % nbuf]).wait()
            return carry

        lax.fori_loop(0, n_outer_w, outer, 0)

        plsc.subcore_barrier()

        @pl.when(sid < _NS - 1)
        def _():
            pltpu.sync_copy(acc.at[pl.ds(sid * rpt_a, rpt_a)],
                            out_hbm.at[cid, pl.ds(sid * rpt_a, rpt_a)])

        @pl.when(sid == _NS - 1)
        def _():
            pltpu.sync_copy(acc.at[pl.ds((_NS - 1) * rpt_a, rpt_b)],
                            out_hbm.at[cid, pl.ds((_NS - 1) * rpt_a, rpt_b)])

    return k(h, edge_r, w_r, zeros_nd)


_BLK = 2000


def _tc_in_proj(x, w1):
    n, kdim = x.shape
    d = w1.shape[0]

    def body(x_ref, w_ref, o_ref):
        o_ref[...] = lax.dot_general(
            x_ref[...], w_ref[...], (((1,), (1,)), ((), ())),
            preferred_element_type=jnp.float32)

    return pl.pallas_call(
        body,
        grid=(n // _BLK,),
        in_specs=[
            pl.BlockSpec((_BLK, kdim), lambda i: (i, 0)),
            pl.BlockSpec((d, kdim), lambda i: (0, 0)),
        ],
        out_specs=pl.BlockSpec((_BLK, d), lambda i: (i, 0)),
        out_shape=jax.ShapeDtypeStruct((n, d), jnp.float32),
    )(x, w1)


def _tc_combine_linear(parts, b, w):
    _, n, d = parts.shape
    d_out = w.shape[0]

    def body(p_ref, b_ref, w_ref, o_ref):
        t = jax.nn.relu(p_ref[0] + p_ref[1] + b_ref[...])
        o_ref[...] = lax.dot_general(
            t, w_ref[...], (((1,), (1,)), ((), ())),
            preferred_element_type=jnp.float32)

    return pl.pallas_call(
        body,
        grid=(n // _BLK,),
        in_specs=[
            pl.BlockSpec((2, _BLK, d), lambda i: (0, i, 0)),
            pl.BlockSpec((1, d), lambda i: (0, 0)),
            pl.BlockSpec((d_out, d), lambda i: (0, 0)),
        ],
        out_specs=pl.BlockSpec((_BLK, d_out), lambda i: (i, 0)),
        out_shape=jax.ShapeDtypeStruct((n, d_out), jnp.float32),
    )(parts, b.reshape(1, d), w)


def _tc_final(parts, b2, w3, b3):
    _, n, d = parts.shape
    d_out = w3.shape[0]

    def body(p_ref, b2_ref, w_ref, b3_ref, o_ref):
        t = jax.nn.relu(p_ref[0] + p_ref[1] + b2_ref[...])
        logits = lax.dot_general(
            t, w_ref[...], (((1,), (1,)), ((), ())),
            preferred_element_type=jnp.float32) + b3_ref[...]
        m = jnp.max(logits, axis=1, keepdims=True)
        e = jnp.exp(logits - m)
        o_ref[...] = e / jnp.sum(e, axis=1, keepdims=True)

    return pl.pallas_call(
        body,
        grid=(n // _BLK,),
        in_specs=[
            pl.BlockSpec((2, _BLK, d), lambda i: (0, i, 0)),
            pl.BlockSpec((1, d), lambda i: (0, 0)),
            pl.BlockSpec((d_out, d), lambda i: (0, 0)),
            pl.BlockSpec((1, d_out), lambda i: (0, 0)),
        ],
        out_specs=pl.BlockSpec((_BLK, d_out), lambda i: (i, 0)),
        out_shape=jax.ShapeDtypeStruct((n, d_out), jnp.float32),
    )(parts, b2.reshape(1, d), w3, b3.reshape(1, d_out))


def kernel(x, edge_index, edge_weight, W1, b1, W2, b2, W3, b3):
    n = x.shape[0]
    e = edge_weight.shape[0]
    c_chunk = 128
    edge_r = edge_index.reshape(2, e // c_chunk, c_chunk)
    w_r = edge_weight.reshape(e // c_chunk, c_chunk)
    zeros_nd = jnp.zeros((n, W1.shape[0]), jnp.float32)

    h1 = _tc_in_proj(x, W1)
    p1 = _sc_edge_aggregate(h1, edge_r, w_r, zeros_nd)
    h2 = _tc_combine_linear(p1, b1, W2)
    p2 = _sc_edge_aggregate(h2, edge_r, w_r, zeros_nd)
    return _tc_final(p2, b2, W3, b3)


# speedup vs baseline: 46.9873x; 1.1018x over previous
import functools

import jax
import jax.numpy as jnp
from jax import lax
from jax.experimental import pallas as pl
from jax.experimental.pallas import tpu as pltpu
from jax.experimental.pallas import tpu_sc as plsc

_NC = 2
_NS = 16
_L = 16


def _sc_edge_aggregate(h, edge_r, w_r, zeros_nd):
    n_nodes, d = h.shape
    n_rows_total, c_chunk = w_r.shape
    nw = _NC * _NS
    stage = 40
    nbuf = 5
    group = stage // nbuf
    total_blocks = n_rows_total // stage
    n_base = total_blocks // nw
    extra = total_blocks % nw
    rpt_a = ((n_nodes // _NS) + 7) // 8 * 8
    rpt_b = n_nodes - (_NS - 1) * rpt_a

    mesh = plsc.VectorSubcoreMesh(core_axis_name="c", subcore_axis_name="s")

    @functools.partial(
        pl.kernel,
        out_type=jax.ShapeDtypeStruct((_NC, n_nodes, d), jnp.float32),
        mesh=mesh,
        scratch_types=[
            pltpu.VMEM((stage, c_chunk), jnp.int32),
            pltpu.VMEM((stage, c_chunk), jnp.int32),
            pltpu.VMEM((stage, c_chunk), jnp.float32),
            pltpu.VMEM((nbuf, c_chunk, d), jnp.float32),
            pltpu.VMEM_SHARED((n_nodes, d), jnp.float32),
            [pltpu.SemaphoreType.DMA] * nbuf,
            [pltpu.SemaphoreType.DMA] * nbuf,
            [pltpu.SemaphoreType.DMA] * 3,
        ],
        compiler_params=pltpu.CompilerParams(use_tc_tiling_on_sc=False),
    )
    def k(h_hbm, edge_hbm, w_hbm, z_hbm, out_hbm,
          src_v, dst_v, w_v, rows_v, acc, sems, ssems, gsems):
        cid = lax.axis_index("c")
        sid = lax.axis_index("s")

        @pl.when(sid < _NS - 1)
        def _():
            pltpu.sync_copy(z_hbm.at[pl.ds(sid * rpt_a, rpt_a)],
                            acc.at[pl.ds(sid * rpt_a, rpt_a)])

        @pl.when(sid == _NS - 1)
        def _():
            pltpu.sync_copy(z_hbm.at[pl.ds((_NS - 1) * rpt_a, rpt_b)],
                            acc.at[pl.ds((_NS - 1) * rpt_a, rpt_b)])

        plsc.subcore_barrier()

        wid = cid * _NS + sid
        n_outer_w = n_base + jnp.where(wid < extra, 1, 0)

        splat_dnums = lax.GatherDimensionNumbers(
            offset_dims=(), collapsed_slice_dims=(0,), start_index_map=(0,))

        def scale_rows(buf, j):
            w16 = None
            cur_w0 = -1
            for r in range(c_chunk):
                w0 = min((r // _L) * _L, c_chunk - _L)
                if w0 != cur_w0:
                    w16 = w_v[j, pl.ds(w0, _L)]
                    cur_w0 = w0
                wsplat = lax.gather(
                    w16, jnp.full((_L, 1), r - w0, jnp.int32),
                    dimension_numbers=splat_dnums, slice_sizes=(1,),
                    mode=lax.GatherScatterMode.PROMISE_IN_BOUNDS)
                rows_v[buf, r, :] = rows_v[buf, r, :] * wsplat

        def issue_gather(j, buf):
            pltpu.async_copy(h_hbm.at[src_v.at[j]], rows_v.at[buf], sems[buf])

        def wait_gather(j, buf):
            pltpu.make_async_copy(h_hbm.at[src_v.at[j]], rows_v.at[buf],
                                  sems[buf]).wait()

        def outer(o, carry):
            row0 = (o * nw + wid) * stage
            c0 = pltpu.async_copy(edge_hbm.at[0, pl.ds(row0, stage)], src_v,
                                  gsems[0])
            c1 = pltpu.async_copy(edge_hbm.at[1, pl.ds(row0, stage)], dst_v,
                                  gsems[1])
            c2 = pltpu.async_copy(w_hbm.at[pl.ds(row0, stage)], w_v, gsems[2])
            c0.wait(); c1.wait(); c2.wait()

            for j0 in range(nbuf - 1):
                issue_gather(j0, j0)

            def inner(g, carry2):
                for k_slot in range(nbuf):
                    j = g * nbuf + k_slot
                    wait_gather(j, k_slot)
                    scale_rows(k_slot, j)
                    pltpu.async_copy(rows_v.at[k_slot],
                                     acc.at[dst_v.at[j]], ssems[k_slot],
                                     add=True)

                    @pl.when(j > 0)
                    def _():
                        pltpu.make_async_copy(
                            rows_v.at[(k_slot + nbuf - 1) % nbuf],
                            acc.at[dst_v.at[j - 1]],
                            ssems[(k_slot + nbuf - 1) % nbuf]).wait()

                    @pl.when(j + nbuf - 1 < stage)
                    def _():
                        issue_gather(j + nbuf - 1, (k_slot + nbuf - 1) % nbuf)
                return carry2

            lax.fori_loop(0, group, inner, 0)
            pltpu.make_async_copy(rows_v.at[(stage - 1) % nbuf],
                                  acc.at[dst_v.at[stage - 1]],
                                  ssems[(stage - 1) % nbuf]).wait()
            return carry

        lax.fori_loop(0, n_outer_w, outer, 0)

        plsc.subcore_barrier()

        @pl.when(sid < _NS - 1)
        def _():
            pltpu.sync_copy(acc.at[pl.ds(sid * rpt_a, rpt_a)],
                            out_hbm.at[cid, pl.ds(sid * rpt_a, rpt_a)])

        @pl.when(sid == _NS - 1)
        def _():
            pltpu.sync_copy(acc.at[pl.ds((_NS - 1) * rpt_a, rpt_b)],
                            out_hbm.at[cid, pl.ds((_NS - 1) * rpt_a, rpt_b)])

    return k(h, edge_r, w_r, zeros_nd)


_BLK_IN = 4000
_BLK = 5000


def _tc_in_proj(x, w1):
    n, kdim = x.shape
    d = w1.shape[0]

    def body(x_ref, w_ref, o_ref):
        o_ref[...] = lax.dot_general(
            x_ref[...], w_ref[...], (((1,), (1,)), ((), ())),
            preferred_element_type=jnp.float32)

    return pl.pallas_call(
        body,
        grid=(n // _BLK_IN,),
        in_specs=[
            pl.BlockSpec((_BLK_IN, kdim), lambda i: (i, 0)),
            pl.BlockSpec((d, kdim), lambda i: (0, 0)),
        ],
        out_specs=pl.BlockSpec((_BLK_IN, d), lambda i: (i, 0)),
        out_shape=jax.ShapeDtypeStruct((n, d), jnp.float32),
    )(x, w1)


def _tc_combine_linear(parts, b, w):
    _, n, d = parts.shape
    d_out = w.shape[0]

    def body(p_ref, b_ref, w_ref, o_ref):
        t = jax.nn.relu(p_ref[0] + p_ref[1] + b_ref[...])
        o_ref[...] = lax.dot_general(
            t, w_ref[...], (((1,), (1,)), ((), ())),
            preferred_element_type=jnp.float32)

    return pl.pallas_call(
        body,
        grid=(n // _BLK,),
        in_specs=[
            pl.BlockSpec((2, _BLK, d), lambda i: (0, i, 0)),
            pl.BlockSpec((1, d), lambda i: (0, 0)),
            pl.BlockSpec((d_out, d), lambda i: (0, 0)),
        ],
        out_specs=pl.BlockSpec((_BLK, d_out), lambda i: (i, 0)),
        out_shape=jax.ShapeDtypeStruct((n, d_out), jnp.float32),
    )(parts, b.reshape(1, d), w)


def _tc_final(parts, b2, w3, b3):
    _, n, d = parts.shape
    d_out = w3.shape[0]

    def body(p_ref, b2_ref, w_ref, b3_ref, o_ref):
        t = jax.nn.relu(p_ref[0] + p_ref[1] + b2_ref[...])
        logits = lax.dot_general(
            t, w_ref[...], (((1,), (1,)), ((), ())),
            preferred_element_type=jnp.float32) + b3_ref[...]
        m = jnp.max(logits, axis=1, keepdims=True)
        e = jnp.exp(logits - m)
        o_ref[...] = e / jnp.sum(e, axis=1, keepdims=True)

    return pl.pallas_call(
        body,
        grid=(n // _BLK,),
        in_specs=[
            pl.BlockSpec((2, _BLK, d), lambda i: (0, i, 0)),
            pl.BlockSpec((1, d), lambda i: (0, 0)),
            pl.BlockSpec((d_out, d), lambda i: (0, 0)),
            pl.BlockSpec((1, d_out), lambda i: (0, 0)),
        ],
        out_specs=pl.BlockSpec((_BLK, d_out), lambda i: (i, 0)),
        out_shape=jax.ShapeDtypeStruct((n, d_out), jnp.float32),
    )(parts, b2.reshape(1, d), w3, b3.reshape(1, d_out))


def kernel(x, edge_index, edge_weight, W1, b1, W2, b2, W3, b3):
    n = x.shape[0]
    e = edge_weight.shape[0]
    c_chunk = 128
    edge_r = edge_index.reshape(2, e // c_chunk, c_chunk)
    w_r = edge_weight.reshape(e // c_chunk, c_chunk)
    zeros_nd = jnp.zeros((n, W1.shape[0]), jnp.float32)

    h1 = _tc_in_proj(x, W1)
    p1 = _sc_edge_aggregate(h1, edge_r, w_r, zeros_nd)
    h2 = _tc_combine_linear(p1, b1, W2)
    p2 = _sc_edge_aggregate(h2, edge_r, w_r, zeros_nd)
    return _tc_final(p2, b2, W3, b3)


# speedup vs baseline: 47.2858x; 1.0064x over previous
import functools

import jax
import jax.numpy as jnp
from jax import lax
from jax.experimental import pallas as pl
from jax.experimental.pallas import tpu as pltpu
from jax.experimental.pallas import tpu_sc as plsc

_NC = 2
_NS = 16
_L = 16


def _sc_edge_aggregate(h, edge_r, w_r, zeros_nd):
    n_nodes, d = h.shape
    n_rows_total, c_chunk = w_r.shape
    nw = _NC * _NS
    stage = 40
    nbuf = 5
    group = stage // nbuf
    total_blocks = n_rows_total // stage
    n_base = total_blocks // nw
    extra = total_blocks % nw
    rpt_a = ((n_nodes // _NS) + 7) // 8 * 8
    rpt_b = n_nodes - (_NS - 1) * rpt_a

    mesh = plsc.VectorSubcoreMesh(core_axis_name="c", subcore_axis_name="s")

    @functools.partial(
        pl.kernel,
        out_type=jax.ShapeDtypeStruct((_NC, n_nodes, d), jnp.float32),
        mesh=mesh,
        scratch_types=[
            pltpu.VMEM((stage, c_chunk), jnp.int32),
            pltpu.VMEM((stage, c_chunk), jnp.int32),
            pltpu.VMEM((stage, c_chunk), jnp.float32),
            pltpu.VMEM((nbuf, c_chunk, d), jnp.float32),
            pltpu.VMEM_SHARED((n_nodes, d), jnp.float32),
            [pltpu.SemaphoreType.DMA] * nbuf,
            [pltpu.SemaphoreType.DMA] * nbuf,
            [pltpu.SemaphoreType.DMA] * 3,
        ],
        compiler_params=pltpu.CompilerParams(use_tc_tiling_on_sc=False),
    )
    def k(h_hbm, edge_hbm, w_hbm, z_hbm, out_hbm,
          src_v, dst_v, w_v, rows_v, acc, sems, ssems, gsems):
        cid = lax.axis_index("c")
        sid = lax.axis_index("s")

        @pl.when(sid < _NS - 1)
        def _():
            pltpu.sync_copy(z_hbm.at[pl.ds(sid * rpt_a, rpt_a)],
                            acc.at[pl.ds(sid * rpt_a, rpt_a)])

        @pl.when(sid == _NS - 1)
        def _():
            pltpu.sync_copy(z_hbm.at[pl.ds((_NS - 1) * rpt_a, rpt_b)],
                            acc.at[pl.ds((_NS - 1) * rpt_a, rpt_b)])

        plsc.subcore_barrier()

        wid = cid * _NS + sid
        n_outer_w = n_base + jnp.where(wid < extra, 1, 0)

        splat_dnums = lax.GatherDimensionNumbers(
            offset_dims=(), collapsed_slice_dims=(0,), start_index_map=(0,))

        def scale_rows(buf, j):
            w16 = None
            cur_w0 = -1
            for r in range(c_chunk):
                w0 = min((r // _L) * _L, c_chunk - _L)
                if w0 != cur_w0:
                    w16 = w_v[j, pl.ds(w0, _L)]
                    cur_w0 = w0
                wsplat = lax.gather(
                    w16, jnp.full((_L, 1), r - w0, jnp.int32),
                    dimension_numbers=splat_dnums, slice_sizes=(1,),
                    mode=lax.GatherScatterMode.PROMISE_IN_BOUNDS)
                rows_v[buf, r, :] = rows_v[buf, r, :] * wsplat

        def issue_gather(j, buf):
            pltpu.async_copy(h_hbm.at[src_v.at[j]], rows_v.at[buf], sems[buf])

        def wait_gather(j, buf):
            pltpu.make_async_copy(h_hbm.at[src_v.at[j]], rows_v.at[buf],
                                  sems[buf]).wait()

        def outer(o, carry):
            row0 = (o * nw + wid) * stage
            c0 = pltpu.async_copy(edge_hbm.at[0, pl.ds(row0, stage)], src_v,
                                  gsems[0])
            c1 = pltpu.async_copy(edge_hbm.at[1, pl.ds(row0, stage)], dst_v,
                                  gsems[1])
            c2 = pltpu.async_copy(w_hbm.at[pl.ds(row0, stage)], w_v, gsems[2])
            c0.wait(); c1.wait(); c2.wait()

            for j0 in range(nbuf - 1):
                issue_gather(j0, j0)

            def inner(g, carry2):
                for k_slot in range(nbuf):
                    j = g * nbuf + k_slot
                    wait_gather(j, k_slot)
                    scale_rows(k_slot, j)
                    pltpu.async_copy(rows_v.at[k_slot],
                                     acc.at[dst_v.at[j]], ssems[k_slot],
                                     add=True)

                    @pl.when(j > 0)
                    def _():
                        pltpu.make_async_copy(
                            rows_v.at[(k_slot + nbuf - 1) % nbuf],
                            acc.at[dst_v.at[j - 1]],
                            ssems[(k_slot + nbuf - 1) % nbuf]).wait()

                    @pl.when(j + nbuf - 1 < stage)
                    def _():
                        issue_gather(j + nbuf - 1, (k_slot + nbuf - 1) % nbuf)
                return carry2

            lax.fori_loop(0, group, inner, 0)
            pltpu.make_async_copy(rows_v.at[(stage - 1) % nbuf],
                                  acc.at[dst_v.at[stage - 1]],
                                  ssems[(stage - 1) % nbuf]).wait()
            return carry

        lax.fori_loop(0, n_outer_w, outer, 0)

        plsc.subcore_barrier()

        @pl.when(sid < _NS - 1)
        def _():
            pltpu.sync_copy(acc.at[pl.ds(sid * rpt_a, rpt_a)],
                            out_hbm.at[cid, pl.ds(sid * rpt_a, rpt_a)])

        @pl.when(sid == _NS - 1)
        def _():
            pltpu.sync_copy(acc.at[pl.ds((_NS - 1) * rpt_a, rpt_b)],
                            out_hbm.at[cid, pl.ds((_NS - 1) * rpt_a, rpt_b)])

    return k(h, edge_r, w_r, zeros_nd)


_BLK_IN = 10000
_BLK = 5000


def _tc_in_proj(x, w1):
    n, kdim = x.shape
    d = w1.shape[0]

    def body(x_ref, w_ref, o_ref):
        o_ref[...] = lax.dot_general(
            x_ref[...], w_ref[...], (((1,), (1,)), ((), ())),
            preferred_element_type=jnp.float32)

    return pl.pallas_call(
        body,
        grid=(n // _BLK_IN,),
        in_specs=[
            pl.BlockSpec((_BLK_IN, kdim), lambda i: (i, 0)),
            pl.BlockSpec((d, kdim), lambda i: (0, 0)),
        ],
        out_specs=pl.BlockSpec((_BLK_IN, d), lambda i: (i, 0)),
        out_shape=jax.ShapeDtypeStruct((n, d), jnp.float32),
    )(x, w1)


def _tc_combine_linear(parts, b, w):
    _, n, d = parts.shape
    d_out = w.shape[0]

    def body(p_ref, b_ref, w_ref, o_ref):
        t = jax.nn.relu(p_ref[0] + p_ref[1] + b_ref[...])
        o_ref[...] = lax.dot_general(
            t, w_ref[...], (((1,), (1,)), ((), ())),
            preferred_element_type=jnp.float32)

    return pl.pallas_call(
        body,
        grid=(n // _BLK,),
        in_specs=[
            pl.BlockSpec((2, _BLK, d), lambda i: (0, i, 0)),
            pl.BlockSpec((1, d), lambda i: (0, 0)),
            pl.BlockSpec((d_out, d), lambda i: (0, 0)),
        ],
        out_specs=pl.BlockSpec((_BLK, d_out), lambda i: (i, 0)),
        out_shape=jax.ShapeDtypeStruct((n, d_out), jnp.float32),
    )(parts, b.reshape(1, d), w)


def _tc_final(parts, b2, w3, b3):
    _, n, d = parts.shape
    d_out = w3.shape[0]

    def body(p_ref, b2_ref, w_ref, b3_ref, o_ref):
        t = jax.nn.relu(p_ref[0] + p_ref[1] + b2_ref[...])
        logits = lax.dot_general(
            t, w_ref[...], (((1,), (1,)), ((), ())),
            preferred_element_type=jnp.float32) + b3_ref[...]
        m = jnp.max(logits, axis=1, keepdims=True)
        e = jnp.exp(logits - m)
        o_ref[...] = e / jnp.sum(e, axis=1, keepdims=True)

    return pl.pallas_call(
        body,
        grid=(n // _BLK,),
        in_specs=[
            pl.BlockSpec((2, _BLK, d), lambda i: (0, i, 0)),
            pl.BlockSpec((1, d), lambda i: (0, 0)),
            pl.BlockSpec((d_out, d), lambda i: (0, 0)),
            pl.BlockSpec((1, d_out), lambda i: (0, 0)),
        ],
        out_specs=pl.BlockSpec((_BLK, d_out), lambda i: (i, 0)),
        out_shape=jax.ShapeDtypeStruct((n, d_out), jnp.float32),
    )(parts, b2.reshape(1, d), w3, b3.reshape(1, d_out))


def kernel(x, edge_index, edge_weight, W1, b1, W2, b2, W3, b3):
    n = x.shape[0]
    e = edge_weight.shape[0]
    c_chunk = 128
    edge_r = edge_index.reshape(2, e // c_chunk, c_chunk)
    w_r = edge_weight.reshape(e // c_chunk, c_chunk)
    zeros_nd = jnp.zeros((n, W1.shape[0]), jnp.float32)

    h1 = _tc_in_proj(x, W1)
    p1 = _sc_edge_aggregate(h1, edge_r, w_r, zeros_nd)
    h2 = _tc_combine_linear(p1, b1, W2)
    p2 = _sc_edge_aggregate(h2, edge_r, w_r, zeros_nd)
    return _tc_final(p2, b2, W3, b3)


# speedup vs baseline: 61.2658x; 1.2956x over previous
import functools

import jax
import jax.numpy as jnp
from jax import lax
from jax.experimental import pallas as pl
from jax.experimental.pallas import tpu as pltpu
from jax.experimental.pallas import tpu_sc as plsc

_NC = 2
_NS = 16
_L = 16


def _sc_edge_aggregate(h, edge_r, w_r, zeros_nd):
    n_nodes, d = h.shape
    n_rows_total, c_chunk = w_r.shape
    nw = _NC * _NS
    stage = 40
    nbuf = 5
    group = stage // nbuf
    total_blocks = n_rows_total // stage
    n_base = total_blocks // nw
    extra = total_blocks % nw
    rpt_a = ((n_nodes // _NS) + 7) // 8 * 8
    rpt_b = n_nodes - (_NS - 1) * rpt_a

    mesh = plsc.VectorSubcoreMesh(core_axis_name="c", subcore_axis_name="s")

    @functools.partial(
        pl.kernel,
        out_type=jax.ShapeDtypeStruct((_NC, n_nodes, d), jnp.float32),
        mesh=mesh,
        scratch_types=[
            pltpu.VMEM((stage, c_chunk), jnp.int32),
            pltpu.VMEM((stage, c_chunk), jnp.int32),
            pltpu.VMEM((stage, c_chunk), jnp.float32),
            pltpu.VMEM((nbuf, c_chunk, d), jnp.float32),
            pltpu.VMEM_SHARED((n_nodes, d), jnp.float32),
            [pltpu.SemaphoreType.DMA] * nbuf,
            [pltpu.SemaphoreType.DMA] * nbuf,
            [pltpu.SemaphoreType.DMA] * 3,
        ],
        compiler_params=pltpu.CompilerParams(use_tc_tiling_on_sc=False),
    )
    def k(h_hbm, edge_hbm, w_hbm, z_hbm, out_hbm,
          src_v, dst_v, w_v, rows_v, acc, sems, ssems, gsems):
        cid = lax.axis_index("c")
        sid = lax.axis_index("s")

        @pl.when(sid < _NS - 1)
        def _():
            pltpu.sync_copy(z_hbm.at[pl.ds(sid * rpt_a, rpt_a)],
                            acc.at[pl.ds(sid * rpt_a, rpt_a)])

        @pl.when(sid == _NS - 1)
        def _():
            pltpu.sync_copy(z_hbm.at[pl.ds((_NS - 1) * rpt_a, rpt_b)],
                            acc.at[pl.ds((_NS - 1) * rpt_a, rpt_b)])

        plsc.subcore_barrier()

        wid = cid * _NS + sid
        n_outer_w = n_base + jnp.where(wid < extra, 1, 0)

        splat_dnums = lax.GatherDimensionNumbers(
            offset_dims=(), collapsed_slice_dims=(0,), start_index_map=(0,))

        def scale_rows(buf, j):
            w16 = None
            cur_w0 = -1
            for r in range(c_chunk):
                w0 = min((r // _L) * _L, c_chunk - _L)
                if w0 != cur_w0:
                    w16 = w_v[j, pl.ds(w0, _L)]
                    cur_w0 = w0
                wsplat = lax.gather(
                    w16, jnp.full((_L, 1), r - w0, jnp.int32),
                    dimension_numbers=splat_dnums, slice_sizes=(1,),
                    mode=lax.GatherScatterMode.PROMISE_IN_BOUNDS)
                rows_v[buf, r, :] = rows_v[buf, r, :] * wsplat

        def issue_gather(j, buf):
            pltpu.async_copy(h_hbm.at[src_v.at[j]], rows_v.at[buf], sems[buf])

        def wait_gather(j, buf):
            pltpu.make_async_copy(h_hbm.at[src_v.at[j]], rows_v.at[buf],
                                  sems[buf]).wait()

        def outer(o, carry):
            row0 = (o * nw + wid) * stage
            c0 = pltpu.async_copy(edge_hbm.at[0, pl.ds(row0, stage)], src_v,
                                  gsems[0])
            c1 = pltpu.async_copy(edge_hbm.at[1, pl.ds(row0, stage)], dst_v,
                                  gsems[1])
            c2 = pltpu.async_copy(w_hbm.at[pl.ds(row0, stage)], w_v, gsems[2])
            c0.wait(); c1.wait(); c2.wait()

            for j0 in range(nbuf - 1):
                issue_gather(j0, j0)

            def inner(g, carry2):
                for k_slot in range(nbuf):
                    j = g * nbuf + k_slot
                    wait_gather(j, k_slot)
                    scale_rows(k_slot, j)
                    pltpu.async_copy(rows_v.at[k_slot],
                                     acc.at[dst_v.at[j]], ssems[k_slot],
                                     add=True)

                    @pl.when(j > 0)
                    def _():
                        pltpu.make_async_copy(
                            rows_v.at[(k_slot + nbuf - 1) % nbuf],
                            acc.at[dst_v.at[j - 1]],
                            ssems[(k_slot + nbuf - 1) % nbuf]).wait()

                    @pl.when(j + nbuf - 1 < stage)
                    def _():
                        issue_gather(j + nbuf - 1, (k_slot + nbuf - 1) % nbuf)
                return carry2

            lax.fori_loop(0, group, inner, 0)
            pltpu.make_async_copy(rows_v.at[(stage - 1) % nbuf],
                                  acc.at[dst_v.at[stage - 1]],
                                  ssems[(stage - 1) % nbuf]).wait()
            return carry

        lax.fori_loop(0, n_outer_w, outer, 0)

        plsc.subcore_barrier()

        @pl.when(sid < _NS - 1)
        def _():
            pltpu.sync_copy(acc.at[pl.ds(sid * rpt_a, rpt_a)],
                            out_hbm.at[cid, pl.ds(sid * rpt_a, rpt_a)])

        @pl.when(sid == _NS - 1)
        def _():
            pltpu.sync_copy(acc.at[pl.ds((_NS - 1) * rpt_a, rpt_b)],
                            out_hbm.at[cid, pl.ds((_NS - 1) * rpt_a, rpt_b)])

    return k(h, edge_r, w_r, zeros_nd)


_BLK_IN = 10000


def _tc_in_proj(x, w1):
    n, kdim = x.shape
    d = w1.shape[0]

    def body(x_ref, w_ref, o_ref):
        o_ref[...] = lax.dot_general(
            x_ref[...], w_ref[...], (((1,), (1,)), ((), ())),
            preferred_element_type=jnp.float32)

    return pl.pallas_call(
        body,
        grid=(n // _BLK_IN,),
        in_specs=[
            pl.BlockSpec((_BLK_IN, kdim), lambda i: (i, 0)),
            pl.BlockSpec((d, kdim), lambda i: (0, 0)),
        ],
        out_specs=pl.BlockSpec((_BLK_IN, d), lambda i: (i, 0)),
        out_shape=jax.ShapeDtypeStruct((n, d), jnp.float32),
    )(x, w1)


def _tc_combine_linear(parts, b_tiled, w_bd):
    _, n8, dp = parts.shape

    def body(p_ref, b_ref, w_ref, o_ref):
        t = jax.nn.relu(p_ref[0] + p_ref[1] + b_ref[...])
        o_ref[...] = jnp.dot(t, w_ref[...],
                             preferred_element_type=jnp.float32)

    return pl.pallas_call(
        body,
        out_shape=jax.ShapeDtypeStruct((n8, dp), jnp.float32),
    )(parts, b_tiled, w_bd)


def _tc_final(parts, b2_tiled, w3_bd, b3_tiled, ones_bd):
    _, n8, dp = parts.shape

    def body(p_ref, b2_ref, w_ref, b3_ref, ones_ref, o_ref):
        t = jax.nn.relu(p_ref[0] + p_ref[1] + b2_ref[...])
        logits = jnp.dot(t, w_ref[...],
                         preferred_element_type=jnp.float32) + b3_ref[...]
        m = jnp.max(logits, axis=1, keepdims=True)
        e = jnp.exp(logits - m)
        s = jnp.dot(e, ones_ref[...], preferred_element_type=jnp.float32)
        o_ref[...] = e / s

    return pl.pallas_call(
        body,
        out_shape=jax.ShapeDtypeStruct((n8, dp), jnp.float32),
    )(parts, b2_tiled, w3_bd, b3_tiled, ones_bd)


def kernel(x, edge_index, edge_weight, W1, b1, W2, b2, W3, b3):
    n = x.shape[0]
    e = edge_weight.shape[0]
    c_chunk = 128
    edge_r = edge_index.reshape(2, e // c_chunk, c_chunk)
    w_r = edge_weight.reshape(e // c_chunk, c_chunk)
    d = W1.shape[0]
    pack = 128 // d
    zeros_nd = jnp.zeros((n, d), jnp.float32)
    eye = jnp.eye(pack, dtype=jnp.float32)

    h1 = _tc_in_proj(x, W1)
    p1 = _sc_edge_aggregate(h1, edge_r, w_r, zeros_nd)
    h2 = _tc_combine_linear(p1.reshape(2, n // pack, pack * d),
                            jnp.tile(b1, pack).reshape(1, pack * d),
                            jnp.kron(eye, W2.T))
    p2 = _sc_edge_aggregate(h2.reshape(n, d), edge_r, w_r, zeros_nd)
    out = _tc_final(p2.reshape(2, n // pack, pack * d),
                    jnp.tile(b2, pack).reshape(1, pack * d),
                    jnp.kron(eye, W3.T),
                    jnp.tile(b3, pack).reshape(1, pack * d),
                    jnp.kron(eye, jnp.ones((d, d), jnp.float32)))
    return out.reshape(n, d)
